# baseline (speedup 1.0000x reference)
import functools

import jax
import jax.numpy as jnp
from jax import lax
from jax.experimental import pallas as pl
from jax.experimental.pallas import tpu as pltpu
from jax.experimental.pallas import tpu_sc as plsc

_DEG = 16
_D = 128
_NSTEP = 8
_NW = 32
_CH = 128
_WEIGHTS = (1.0, 0.6, 0.4, 0.25, 0.1)


def _make_walk_kernel(n_pos):
    ppw = n_pos // _NW
    nch = ppw // _CH
    mesh = plsc.VectorSubcoreMesh(core_axis_name="c", subcore_axis_name="s")

    @functools.partial(
        pl.kernel,
        out_type=jax.ShapeDtypeStruct((n_pos, _D), jnp.float32),
        mesh=mesh,
        scratch_types=[
            pltpu.VMEM((5, _CH), jnp.int32),
            pltpu.VMEM((_CH,), jnp.int32),
            pltpu.VMEM((_CH,), jnp.int32),
            pltpu.VMEM((_NSTEP, _CH), jnp.int32),
            pltpu.VMEM((5, _CH, _D), jnp.float32),
            pltpu.VMEM((_CH, _D), jnp.float32),
            pltpu.SemaphoreType.DMA,
        ],
    )
    def walk(seq_hbm, tqk_hbm, tkq_hbm, tqs_hbm, tsq_hbm, ch_hbm, eq_hbm,
             out_hbm, teven, todd, idxb, chb, rows, outb, sem):
        nc = 2
        wid = lax.axis_index("s") * nc + lax.axis_index("c")
        tabs = (tqk_hbm, tkq_hbm, tqs_hbm, tsq_hbm) * 2
        for c in range(nch):
            base = wid * ppw + c * _CH
            pltpu.sync_copy(seq_hbm.at[pl.ds(base, _CH)], teven.at[0])
            pltpu.sync_copy(ch_hbm.at[:, pl.ds(base, _CH)], chb)
            for step in range(_NSTEP):
                src = teven if step % 2 == 0 else todd
                for j in range(_CH // 16):
                    s = pl.ds(j * 16, 16)
                    cur = src[step // 2, s] if step % 2 == 0 else src[s]
                    idxb[s] = cur * _DEG + chb[step, s]
                dst = todd if step % 2 == 0 else teven.at[step // 2 + 1]
                pltpu.async_copy(tabs[step].at[idxb], dst, sem).wait()
            for k in range(5):
                pltpu.async_copy(eq_hbm.at[teven.at[k]], rows.at[k], sem).wait()

            def body(i, carry):
                for j in range(_D // 16):
                    s = pl.ds(j * 16, 16)
                    acc = rows[0, i, s] * _WEIGHTS[0]
                    for k in range(1, 5):
                        acc = acc + rows[k, i, s] * _WEIGHTS[k]
                    outb[i, s] = acc
                return carry

            lax.fori_loop(0, _CH, body, 0)
            pltpu.sync_copy(outb, out_hbm.at[pl.ds(base, _CH), :])

    return walk


def kernel(x_question, y_knowledge, seq_q, E_q, E_kc, E_stu,
           nbr_q_kc, nbr_kc_q, nbr_q_stu, nbr_stu_q, choices):
    bs, seq_len = seq_q.shape
    n_pos = bs * seq_len
    walk = _make_walk_kernel(n_pos)
    hq = walk(
        seq_q.reshape(-1),
        nbr_q_kc.reshape(-1),
        nbr_kc_q.reshape(-1),
        nbr_q_stu.reshape(-1),
        nbr_stu_q.reshape(-1),
        choices,
        E_q,
    )
    hq = hq.reshape(bs, seq_len, _D)
    return (hq, hq)


# speedup vs baseline: 1.2376x; 1.2376x over previous
import functools

import jax
import jax.numpy as jnp
from jax import lax
from jax.experimental import pallas as pl
from jax.experimental.pallas import tpu as pltpu
from jax.experimental.pallas import tpu_sc as plsc

_DEG = 16
_D = 128
_NSTEP = 8
_NW = 32
_CH = 128
_ECH = 64
_WEIGHTS = (1.0, 0.6, 0.4, 0.25, 0.1)


def _make_walk_kernel(n_pos):
    ppw = n_pos // _NW
    nch = ppw // _CH
    nech = ppw // _ECH
    mesh = plsc.VectorSubcoreMesh(core_axis_name="c", subcore_axis_name="s")

    @functools.partial(
        pl.kernel,
        out_type=jax.ShapeDtypeStruct((n_pos, _D), jnp.float32),
        mesh=mesh,
        scratch_types=[
            pltpu.VMEM((5, nch, _CH), jnp.int32),
            pltpu.VMEM((nch, _CH), jnp.int32),
            pltpu.VMEM((nch, _CH), jnp.int32),
            pltpu.VMEM((nch, _NSTEP, _CH), jnp.int32),
            pltpu.VMEM((2, 5, _ECH, _D), jnp.float32),
            pltpu.VMEM((2, _ECH, _D), jnp.float32),
            pltpu.SemaphoreType.DMA,
            pltpu.SemaphoreType.DMA,
            pltpu.SemaphoreType.DMA,
            pltpu.SemaphoreType.DMA,
            pltpu.SemaphoreType.DMA,
        ],
    )
    def walk(seq_hbm, tqk_hbm, tkq_hbm, tqs_hbm, tsq_hbm, ch_hbm, eq_hbm,
             out_hbm, teven, todd, idxb, chb, rows, outb,
             sem_w, sem_g0, sem_g1, sem_o0, sem_o1, *, nc=2):
        wid = lax.axis_index("s") * nc + lax.axis_index("c")
        tabs = (tqk_hbm, tkq_hbm, tqs_hbm, tsq_hbm) * 2

        descs = []
        for c in range(nch):
            base = wid * ppw + c * _CH
            descs.append(pltpu.async_copy(
                seq_hbm.at[pl.ds(base, _CH)], teven.at[0, c], sem_w))
            descs.append(pltpu.async_copy(
                ch_hbm.at[:, pl.ds(base, _CH)], chb.at[c], sem_w))
        for d in descs:
            d.wait()

        for step in range(_NSTEP):
            k = step // 2
            even = step % 2 == 0

            @plsc.parallel_loop(0, nch)
            def _(c):
                for j in range(_CH // 16):
                    s = pl.ds(j * 16, 16)
                    cur = teven[k, c, s] if even else todd[c, s]
                    idxb[c, s] = cur * _DEG + chb[c, step, s]

            descs = []
            for c in range(nch):
                dst = todd.at[c] if even else teven.at[k + 1, c]
                descs.append(pltpu.async_copy(
                    tabs[step].at[idxb.at[c]], dst, sem_w))
            for d in descs:
                d.wait()

        def fire(e):
            b = e % 2
            sem = sem_g0 if b == 0 else sem_g1
            c, half = divmod(e, 2)
            return [pltpu.async_copy(
                eq_hbm.at[teven.at[kk, c, pl.ds(half * _ECH, _ECH)]],
                rows.at[b, kk], sem)
                for kk in range(5)]

        g_descs = {0: fire(0)}
        o_descs = {}
        for e in range(nech):
            b = e % 2
            if e + 1 < nech:
                g_descs[e + 1] = fire(e + 1)
            for d in g_descs.pop(e):
                d.wait()
            if e - 2 in o_descs:
                o_descs.pop(e - 2).wait()

            @plsc.parallel_loop(0, _ECH)
            def _(i):
                for j in range(_D // 16):
                    s = pl.ds(j * 16, 16)
                    acc = rows[b, 0, i, s] * _WEIGHTS[0]
                    for kk in range(1, 5):
                        acc = acc + rows[b, kk, i, s] * _WEIGHTS[kk]
                    outb[b, i, s] = acc

            pos = wid * ppw + e * _ECH
            sem = sem_o0 if b == 0 else sem_o1
            o_descs[e] = pltpu.async_copy(
                outb.at[b], out_hbm.at[pl.ds(pos, _ECH), :], sem)
        for d in o_descs.values():
            d.wait()

    return walk


def kernel(x_question, y_knowledge, seq_q, E_q, E_kc, E_stu,
           nbr_q_kc, nbr_kc_q, nbr_q_stu, nbr_stu_q, choices):
    bs, seq_len = seq_q.shape
    n_pos = bs * seq_len
    walk = _make_walk_kernel(n_pos)
    hq = walk(
        seq_q.reshape(-1),
        nbr_q_kc.reshape(-1),
        nbr_kc_q.reshape(-1),
        nbr_q_stu.reshape(-1),
        nbr_stu_q.reshape(-1),
        choices,
        E_q,
    )
    hq = hq.reshape(bs, seq_len, _D)
    return (hq, hq)
